# baseline (speedup 1.0000x reference)
import jax
import jax.numpy as jnp
from jax.experimental import pallas as pl
from jax.experimental.pallas import tpu as pltpu

_CONF = 0.01
_BLK = 2000


def _split_body(pred_ref, ind_ref, praw_ref):
    pred = pred_ref[0]
    ind_ref[0] = pred[:, 4:] > _CONF
    praw_ref[0] = pred[:, :4]


def _decode_body(p_ref, d_ref, out_ref):
    d = d_ref[...]
    p = p_ref[0]
    u = 0.1 * p
    e = 0.5 * jnp.exp(0.2 * p)
    lo = d + jnp.roll(d, -2, axis=-1) * (u - jnp.roll(e, -2, axis=-1))
    hi = jnp.roll(d, 2, axis=-1) + d * (jnp.roll(u, 2, axis=-1) + e)
    lane = jax.lax.broadcasted_iota(jnp.int32, d.shape, 1)
    out_ref[0] = jnp.where((lane & 2) == 0, lo, hi)


def kernel(predicts, dboxes):
    b, n, c = predicts.shape
    nblk = n // _BLK
    ind, praw = pl.pallas_call(
        _split_body,
        grid=(b, nblk),
        in_specs=[pl.BlockSpec((1, _BLK, c), lambda i, j: (i, j, 0))],
        out_specs=[
            pl.BlockSpec((1, _BLK, c - 4), lambda i, j: (i, j, 0)),
            pl.BlockSpec((1, _BLK, 4), lambda i, j: (i, j, 0)),
        ],
        out_shape=[
            jax.ShapeDtypeStruct((b, n, c - 4), jnp.bool_),
            jax.ShapeDtypeStruct((b, n, 4), jnp.float32),
        ],
        compiler_params=pltpu.CompilerParams(
            dimension_semantics=("parallel", "parallel"),
        ),
    )(predicts)

    rows = n * 4 // 128
    p_flat = praw.reshape(b, rows, 128)
    d_flat = dboxes.reshape(rows, 128)
    loc = pl.pallas_call(
        _decode_body,
        grid=(b,),
        in_specs=[
            pl.BlockSpec((1, rows, 128), lambda i: (i, 0, 0)),
            pl.BlockSpec((rows, 128), lambda i: (0, 0)),
        ],
        out_specs=pl.BlockSpec((1, rows, 128), lambda i: (i, 0, 0)),
        out_shape=jax.ShapeDtypeStruct((b, rows, 128), jnp.float32),
        compiler_params=pltpu.CompilerParams(
            dimension_semantics=("arbitrary",),
        ),
    )(p_flat, d_flat)
    return loc.reshape(b, n, 4), ind


# speedup vs baseline: 6.4352x; 6.4352x over previous
import jax
import jax.numpy as jnp
from jax.experimental import pallas as pl
from jax.experimental.pallas import tpu as pltpu

_CONF = 0.01


def _ind_body(pred_ref, ind_ref):
    ind_ref[...] = (pred_ref[...] > _CONF).astype(jnp.int8)


def _decode_body(p_ref, d_ref, loc_ref):
    p = p_ref[...]
    d = d_ref[...]
    d0, d1, d2, d3 = d[0:1], d[1:2], d[2:3], d[3:4]
    u0 = 0.1 * p[0]
    u1 = 0.1 * p[1]
    e2 = 0.5 * jnp.exp(0.2 * p[2])
    e3 = 0.5 * jnp.exp(0.2 * p[3])
    loc_ref[:, 0, :] = d0 + d2 * (u0 - e2)
    loc_ref[:, 1, :] = d1 + d3 * (u1 - e3)
    loc_ref[:, 2, :] = d0 + d2 * (u0 + e2)
    loc_ref[:, 3, :] = d1 + d3 * (u1 + e3)


def kernel(predicts, dboxes):
    b, n, c = predicts.shape
    pt = jnp.transpose(predicts, (2, 0, 1))
    dt = jnp.transpose(dboxes, (1, 0))

    ind_t = pl.pallas_call(
        _ind_body,
        grid=(c - 4,),
        in_specs=[pl.BlockSpec((1, b, n), lambda j: (j + 4, 0, 0))],
        out_specs=pl.BlockSpec((1, b, n), lambda j: (j, 0, 0)),
        out_shape=jax.ShapeDtypeStruct((c - 4, b, n), jnp.int8),
        compiler_params=pltpu.CompilerParams(
            dimension_semantics=("parallel",),
        ),
    )(pt)

    loc_t = pl.pallas_call(
        _decode_body,
        grid=(1,),
        in_specs=[
            pl.BlockSpec((4, b, n), lambda j: (0, 0, 0)),
            pl.BlockSpec((4, n), lambda j: (0, 0)),
        ],
        out_specs=pl.BlockSpec((b, 4, n), lambda j: (0, 0, 0)),
        out_shape=jax.ShapeDtypeStruct((b, 4, n), jnp.float32),
        compiler_params=pltpu.CompilerParams(
            dimension_semantics=("arbitrary",),
        ),
    )(pt, dt)

    loc = jnp.transpose(loc_t, (0, 2, 1))
    ind = jnp.transpose(ind_t, (1, 2, 0)).astype(jnp.bool_)
    return loc, ind
